# Initial kernel scaffold; baseline (speedup 1.0000x reference)
import jax
import jax.numpy as jnp
from jax.experimental import pallas as pl


def kernel(pq, fq, ps, fs, params):
    raise NotImplementedError("write your pallas kernel here")


# speedup vs baseline: 1.6002x; 1.6002x over previous
import functools

import jax
import jax.numpy as jnp
from jax.experimental import pallas as pl
from jax.experimental.pallas import tpu as pltpu

_DOWN_RATES = [1, 4, 2]
_KNNS = [16, 12, 8]


def _fps_body(x_ref, y_ref, z_ref, out_ref, *, npoint):
    x = x_ref[...]
    y = y_ref[...]
    z = z_ref[...]
    bc, n = x.shape
    iota = jax.lax.broadcasted_iota(jnp.int32, (bc, n), 1)
    iota_out = jax.lax.broadcasted_iota(jnp.int32, (bc, npoint), 1)

    def step(i, carry):
        dists, far, acc = carry
        acc = jnp.where(iota_out == i, far, acc)
        oh = (iota == far).astype(jnp.float32)
        cx = jnp.sum(x * oh, axis=1, keepdims=True)
        cy = jnp.sum(y * oh, axis=1, keepdims=True)
        cz = jnp.sum(z * oh, axis=1, keepdims=True)
        dx = x - cx
        dy = y - cy
        dz = z - cz
        d = dx * dx + dy * dy + dz * dz
        dists = jnp.minimum(dists, d)
        m = jnp.max(dists, axis=1, keepdims=True)
        far_new = jnp.min(jnp.where(dists == m, iota, n), axis=1,
                          keepdims=True).astype(jnp.int32)
        return dists, far_new, acc

    dists0 = jnp.full((bc, n), 1e10, jnp.float32)
    far0 = jnp.zeros((bc, 1), jnp.int32)
    acc0 = out_ref[...]
    _, _, acc = jax.lax.fori_loop(0, npoint, step, (dists0, far0, acc0))
    out_ref[...] = acc


def _fps_batched(xyz_all, npoint):
    bc, n, _ = xyz_all.shape
    x = xyz_all[:, :, 0]
    y = xyz_all[:, :, 1]
    z = xyz_all[:, :, 2]
    return pl.pallas_call(
        functools.partial(_fps_body, npoint=npoint),
        out_shape=jax.ShapeDtypeStruct((bc, npoint), jnp.int32),
    )(x, y, z)


def _conv1d(x, w, b):
    return jnp.einsum('oc,bcn->bon', w, x) + b[None, :, None]


def _conv2d(x, w, b):
    return jnp.einsum('oc,bcnk->bonk', w, x) + b[None, :, None, None]


def _bn2d(x, gamma, beta, eps=1e-5):
    mean = jnp.mean(x, axis=(0, 2, 3), keepdims=True)
    var = jnp.var(x, axis=(0, 2, 3), keepdims=True)
    xhat = (x - mean) / jnp.sqrt(var + eps)
    return gamma[None, :, None, None] * xhat + beta[None, :, None, None]


def _square_distance(a, b):
    return (jnp.sum(a * a, -1)[:, :, None] + jnp.sum(b * b, -1)[:, None, :]
            - 2.0 * jnp.einsum('bnd,bmd->bnm', a, b))


def _gather(feat, idx):
    return jax.vmap(lambda f, i: f[:, i])(feat, idx)


def _query_knn(k, support_xyz, query_xyz):
    d = _square_distance(query_xyz, support_xyz)
    _, idx = jax.lax.top_k(-d, k)
    return idx


def _three_inter(f, p1, p2):
    d = _square_distance(jnp.transpose(p2, (0, 2, 1)), jnp.transpose(p1, (0, 2, 1)))
    negv, idx = jax.lax.top_k(-d, 3)
    dis = -negv
    dist_recip = 1.0 / (dis + 1e-8)
    weight = dist_recip / jnp.sum(dist_recip, axis=2, keepdims=True)
    nb = _gather(f, idx)
    return jnp.sum(nb * weight[:, None, :, :], axis=-1)


def _mlp_res(x, p):
    shortcut = _conv1d(x, p['ws'], p['bs'])
    h = jax.nn.relu(_conv1d(x, p['w1'], p['b1']))
    return _conv1d(h, p['w2'], p['b2']) + shortcut


def _vector_attention(p, pq, fq, ps, fs, n_knn):
    identity = fq
    q = _conv1d(fq, p['wq'], p['bq'])
    k = _conv1d(fs, p['wk'], p['bk'])
    v = _conv1d(fs, p['wv'], p['bv'])
    idx = _query_knn(n_knn, jnp.transpose(ps, (0, 2, 1)), jnp.transpose(pq, (0, 2, 1)))
    key = _gather(k, idx)
    qk_rel = q[:, :, :, None] - key
    pos_rel = pq[:, :, :, None] - _gather(ps, idx)
    pe = _conv2d(pos_rel, p['pw1'], p['pb1'])
    pe = jax.nn.relu(_bn2d(pe, p['pg1'], p['pbt1']))
    pe = _conv2d(pe, p['pw2'], p['pb2'])
    a = _conv2d(qk_rel + pe, p['aw1'], p['ab1'])
    a = jax.nn.relu(_bn2d(a, p['ag1'], p['abt1']))
    a = _conv2d(a, p['aw2'], p['ab2'])
    a = jax.nn.softmax(a, axis=-1)
    val = _gather(v, idx) + pe
    agg = jnp.sum(a * val, axis=-1)
    return _conv1d(agg, p['we'], p['be']) + identity


def kernel(pq, fq, ps, fs, params):
    b, _, n = pq.shape
    xyz_all = jnp.concatenate(
        [jnp.transpose(pq, (0, 2, 1)), jnp.transpose(ps, (0, 2, 1))], axis=0)
    idx512 = _fps_batched(xyz_all, n // 4)
    fq512, fs512 = idx512[:b], idx512[b:]
    fq256, fs256 = fq512[:, :n // 8], fs512[:, :n // 8]
    fps_q = [None, fq512, fq256]
    fps_s = [None, fs512, fs256]

    num_scale = len(_DOWN_RATES)
    pre_f = None
    pre_pos = None
    for i in range(num_scale - 1, -1, -1):
        pos1 = pq if fps_q[i] is None else _gather(pq, fps_q[i])
        pos2 = ps if fps_s[i] is None else _gather(ps, fps_s[i])
        f1 = fq if fps_q[i] is None else _gather(fq, fps_q[i])
        f2 = fs if fps_s[i] is None else _gather(fs, fps_s[i])
        if i != num_scale - 1:
            proj1 = _three_inter(pre_f, pre_pos, pos1)
            proj2 = _three_inter(pre_f, pre_pos, pos2)
            f1 = _mlp_res(jnp.concatenate([f1, proj1], axis=1), params['qmlp'][i])
            f2 = _mlp_res(jnp.concatenate([f2, proj2], axis=1), params['smlp'][i])
        pre_f = _vector_attention(params['attn'][i], pos1, f1, pos2, f2, _KNNS[i])
        pre_pos = pos1
    return (pre_f, fps_q[1], fps_q[2], fps_s[1], fps_s[2])


# speedup vs baseline: 2.4221x; 1.5136x over previous
import functools

import jax
import jax.numpy as jnp
from jax.experimental import pallas as pl
from jax.experimental.pallas import tpu as pltpu

_DOWN_RATES = [1, 4, 2]
_KNNS = [16, 12, 8]


def _fps_body(x_ref, y_ref, z_ref, out_ref, *, npoint):
    x = x_ref[...]
    y = y_ref[...]
    z = z_ref[...]
    bc, n = x.shape
    iota = jax.lax.broadcasted_iota(jnp.int32, (bc, n), 1)
    iota_out = jax.lax.broadcasted_iota(jnp.int32, (bc, npoint), 1)

    def step(i, carry):
        dists, far, acc = carry
        acc = jnp.where(iota_out == i, far, acc)
        oh = (iota == far).astype(jnp.float32)
        cx = jnp.sum(x * oh, axis=1, keepdims=True)
        cy = jnp.sum(y * oh, axis=1, keepdims=True)
        cz = jnp.sum(z * oh, axis=1, keepdims=True)
        dx = x - cx
        dy = y - cy
        dz = z - cz
        d = dx * dx + dy * dy + dz * dz
        dists = jnp.minimum(dists, d)
        m = jnp.max(dists, axis=1, keepdims=True)
        far_new = jnp.min(jnp.where(dists == m, iota, n), axis=1,
                          keepdims=True).astype(jnp.int32)
        return dists, far_new, acc

    dists0 = jnp.full((bc, n), 1e10, jnp.float32)
    far0 = jnp.zeros((bc, 1), jnp.int32)
    acc0 = out_ref[...]
    _, _, acc = jax.lax.fori_loop(0, npoint, step, (dists0, far0, acc0))
    out_ref[...] = acc


def _fps_batched(xyz_all, npoint):
    bc, n, _ = xyz_all.shape
    x = xyz_all[:, :, 0]
    y = xyz_all[:, :, 1]
    z = xyz_all[:, :, 2]
    return pl.pallas_call(
        functools.partial(_fps_body, npoint=npoint),
        out_shape=jax.ShapeDtypeStruct((bc, npoint), jnp.int32),
    )(x, y, z)


def _topk_body(st_ref, q_ref, idx_ref, val_ref, *, k, ns):
    st = st_ref[0]
    q3 = q_ref[0]
    sx, sy, sz = st[:, 0:1], st[:, 1:2], st[:, 2:3]
    qx, qy, qz = q3[0:1, :], q3[1:2, :], q3[2:3, :]
    sn = (sx * sx + sy * sy) + sz * sz
    qn = (qx * qx + qy * qy) + qz * qz
    def _b(v):
        return v.astype(jnp.bfloat16).astype(jnp.float32)
    cross = (_b(sx) * _b(qx) + _b(sy) * _b(qy)) + _b(sz) * _b(qz)
    d = qn + sn - 2.0 * cross
    iota0 = jax.lax.broadcasted_iota(jnp.int32, d.shape, 0)
    idxs = []
    vals = []
    for _ in range(k):
        m = jnp.min(d, axis=0, keepdims=True)
        c = jnp.min(jnp.where(d == m, iota0, ns), axis=0, keepdims=True)
        idxs.append(c)
        vals.append(m)
        d = jnp.where(iota0 == c, jnp.inf, d)
    idx_ref[0] = jnp.concatenate(idxs, axis=0)
    val_ref[0] = jnp.concatenate(vals, axis=0)


def _knn_topk(support_t, query, k, tile=256):
    b, ns, _ = support_t.shape
    _, _, nq = query.shape
    tile = min(tile, nq)
    grid = (b, nq // tile)
    kern = pl.pallas_call(
        functools.partial(_topk_body, k=k, ns=ns),
        grid=grid,
        in_specs=[
            pl.BlockSpec((1, ns, 3), lambda bb, t: (bb, 0, 0)),
            pl.BlockSpec((1, 3, tile), lambda bb, t: (bb, 0, t)),
        ],
        out_specs=[
            pl.BlockSpec((1, k, tile), lambda bb, t: (bb, 0, t)),
            pl.BlockSpec((1, k, tile), lambda bb, t: (bb, 0, t)),
        ],
        out_shape=[
            jax.ShapeDtypeStruct((b, k, nq), jnp.int32),
            jax.ShapeDtypeStruct((b, k, nq), jnp.float32),
        ],
    )
    return kern(support_t, query)


def _conv1d(x, w, b):
    return jnp.einsum('oc,bcn->bon', w, x) + b[None, :, None]


def _conv2d(x, w, b):
    return jnp.einsum('oc,bcnk->bonk', w, x) + b[None, :, None, None]


def _bn2d(x, gamma, beta, eps=1e-5):
    mean = jnp.mean(x, axis=(0, 2, 3), keepdims=True)
    var = jnp.var(x, axis=(0, 2, 3), keepdims=True)
    xhat = (x - mean) / jnp.sqrt(var + eps)
    return gamma[None, :, None, None] * xhat + beta[None, :, None, None]


def _square_distance(a, b):
    return (jnp.sum(a * a, -1)[:, :, None] + jnp.sum(b * b, -1)[:, None, :]
            - 2.0 * jnp.einsum('bnd,bmd->bnm', a, b))


def _gather(feat, idx):
    return jax.vmap(lambda f, i: f[:, i])(feat, idx)


def _three_inter(f, p1, p2):
    idx, val = _knn_topk(jnp.transpose(p1, (0, 2, 1)), p2, 3)
    dis = jnp.transpose(val, (0, 2, 1))
    dist_recip = 1.0 / (dis + 1e-8)
    weight = dist_recip / jnp.sum(dist_recip, axis=2, keepdims=True)
    nb = _gather(f, jnp.transpose(idx, (0, 2, 1)))
    return jnp.sum(nb * weight[:, None, :, :], axis=-1)


def _mlp_res(x, p):
    shortcut = _conv1d(x, p['ws'], p['bs'])
    h = jax.nn.relu(_conv1d(x, p['w1'], p['b1']))
    return _conv1d(h, p['w2'], p['b2']) + shortcut


def _vector_attention(p, pq, fq, ps, fs, n_knn):
    identity = fq
    q = _conv1d(fq, p['wq'], p['bq'])
    k = _conv1d(fs, p['wk'], p['bk'])
    v = _conv1d(fs, p['wv'], p['bv'])
    idx_t, _ = _knn_topk(jnp.transpose(ps, (0, 2, 1)), pq, n_knn)
    idx = jnp.transpose(idx_t, (0, 2, 1))
    key = _gather(k, idx)
    qk_rel = q[:, :, :, None] - key
    pos_rel = pq[:, :, :, None] - _gather(ps, idx)
    pe = _conv2d(pos_rel, p['pw1'], p['pb1'])
    pe = jax.nn.relu(_bn2d(pe, p['pg1'], p['pbt1']))
    pe = _conv2d(pe, p['pw2'], p['pb2'])
    a = _conv2d(qk_rel + pe, p['aw1'], p['ab1'])
    a = jax.nn.relu(_bn2d(a, p['ag1'], p['abt1']))
    a = _conv2d(a, p['aw2'], p['ab2'])
    a = jax.nn.softmax(a, axis=-1)
    val = _gather(v, idx) + pe
    agg = jnp.sum(a * val, axis=-1)
    return _conv1d(agg, p['we'], p['be']) + identity


def kernel(pq, fq, ps, fs, params):
    b, _, n = pq.shape
    xyz_all = jnp.concatenate(
        [jnp.transpose(pq, (0, 2, 1)), jnp.transpose(ps, (0, 2, 1))], axis=0)
    idx512 = _fps_batched(xyz_all, n // 4)
    fq512, fs512 = idx512[:b], idx512[b:]
    fq256, fs256 = fq512[:, :n // 8], fs512[:, :n // 8]
    fps_q = [None, fq512, fq256]
    fps_s = [None, fs512, fs256]

    num_scale = len(_DOWN_RATES)
    pre_f = None
    pre_pos = None
    for i in range(num_scale - 1, -1, -1):
        pos1 = pq if fps_q[i] is None else _gather(pq, fps_q[i])
        pos2 = ps if fps_s[i] is None else _gather(ps, fps_s[i])
        f1 = fq if fps_q[i] is None else _gather(fq, fps_q[i])
        f2 = fs if fps_s[i] is None else _gather(fs, fps_s[i])
        if i != num_scale - 1:
            proj1 = _three_inter(pre_f, pre_pos, pos1)
            proj2 = _three_inter(pre_f, pre_pos, pos2)
            f1 = _mlp_res(jnp.concatenate([f1, proj1], axis=1), params['qmlp'][i])
            f2 = _mlp_res(jnp.concatenate([f2, proj2], axis=1), params['smlp'][i])
        pre_f = _vector_attention(params['attn'][i], pos1, f1, pos2, f2, _KNNS[i])
        pre_pos = pos1
    return (pre_f, fps_q[1], fps_q[2], fps_s[1], fps_s[2])


# speedup vs baseline: 8.5900x; 3.5465x over previous
import functools

import jax
import jax.numpy as jnp
from jax.experimental import pallas as pl
from jax.experimental.pallas import tpu as pltpu

_DOWN_RATES = [1, 4, 2]
_KNNS = [16, 12, 8]


def _fps_body(x_ref, y_ref, z_ref, out_ref, *, npoint):
    x = x_ref[...]
    y = y_ref[...]
    z = z_ref[...]
    bc, n = x.shape
    iota = jax.lax.broadcasted_iota(jnp.int32, (bc, n), 1)
    iota_out = jax.lax.broadcasted_iota(jnp.int32, (bc, npoint), 1)

    def step(i, carry):
        dists, far, acc = carry
        acc = jnp.where(iota_out == i, far, acc)
        oh = (iota == far).astype(jnp.float32)
        cx = jnp.sum(x * oh, axis=1, keepdims=True)
        cy = jnp.sum(y * oh, axis=1, keepdims=True)
        cz = jnp.sum(z * oh, axis=1, keepdims=True)
        dx = x - cx
        dy = y - cy
        dz = z - cz
        d = dx * dx + dy * dy + dz * dz
        dists = jnp.minimum(dists, d)
        m = jnp.max(dists, axis=1, keepdims=True)
        far_new = jnp.min(jnp.where(dists == m, iota, n), axis=1,
                          keepdims=True).astype(jnp.int32)
        return dists, far_new, acc

    dists0 = jnp.full((bc, n), 1e10, jnp.float32)
    far0 = jnp.zeros((bc, 1), jnp.int32)
    acc0 = out_ref[...]
    _, _, acc = jax.lax.fori_loop(0, npoint, step, (dists0, far0, acc0))
    out_ref[...] = acc


def _fps_batched(xyz_all, npoint):
    bc, n, _ = xyz_all.shape
    x = xyz_all[:, :, 0]
    y = xyz_all[:, :, 1]
    z = xyz_all[:, :, 2]
    return pl.pallas_call(
        functools.partial(_fps_body, npoint=npoint),
        out_shape=jax.ShapeDtypeStruct((bc, npoint), jnp.int32),
    )(x, y, z)


def _topk_body(st_ref, q_ref, idx_ref, val_ref, *, k, ns):
    st = st_ref[0]
    q3 = q_ref[0]
    sx, sy, sz = st[:, 0:1], st[:, 1:2], st[:, 2:3]
    qx, qy, qz = q3[0:1, :], q3[1:2, :], q3[2:3, :]
    sn = (sx * sx + sy * sy) + sz * sz
    qn = (qx * qx + qy * qy) + qz * qz
    def _b(v):
        return v.astype(jnp.bfloat16).astype(jnp.float32)
    cross = (_b(sx) * _b(qx) + _b(sy) * _b(qy)) + _b(sz) * _b(qz)
    d = qn + sn - 2.0 * cross
    iota0 = jax.lax.broadcasted_iota(jnp.int32, d.shape, 0)
    idxs = []
    vals = []
    for _ in range(k):
        m = jnp.min(d, axis=0, keepdims=True)
        c = jnp.min(jnp.where(d == m, iota0, ns), axis=0, keepdims=True)
        idxs.append(c)
        vals.append(m)
        d = jnp.where(iota0 == c, jnp.inf, d)
    idx_ref[0] = jnp.concatenate(idxs, axis=0)
    val_ref[0] = jnp.concatenate(vals, axis=0)


def _knn_topk(support_t, query, k, tile=256):
    b, ns, _ = support_t.shape
    _, _, nq = query.shape
    tile = min(tile, nq)
    grid = (b, nq // tile)
    kern = pl.pallas_call(
        functools.partial(_topk_body, k=k, ns=ns),
        grid=grid,
        in_specs=[
            pl.BlockSpec((1, ns, 3), lambda bb, t: (bb, 0, 0)),
            pl.BlockSpec((1, 3, tile), lambda bb, t: (bb, 0, t)),
        ],
        out_specs=[
            pl.BlockSpec((1, k, tile), lambda bb, t: (bb, 0, t)),
            pl.BlockSpec((1, k, tile), lambda bb, t: (bb, 0, t)),
        ],
        out_shape=[
            jax.ShapeDtypeStruct((b, k, nq), jnp.int32),
            jax.ShapeDtypeStruct((b, k, nq), jnp.float32),
        ],
    )
    return kern(support_t, query)


def _conv1d(x, w, b):
    return jnp.einsum('oc,bcn->bon', w, x) + b[None, :, None]


def _conv2d(x, w, b):
    return jnp.einsum('oc,bcnk->bonk', w, x) + b[None, :, None, None]


def _bn2d(x, gamma, beta, eps=1e-5):
    mean = jnp.mean(x, axis=(0, 2, 3), keepdims=True)
    var = jnp.var(x, axis=(0, 2, 3), keepdims=True)
    xhat = (x - mean) / jnp.sqrt(var + eps)
    return gamma[None, :, None, None] * xhat + beta[None, :, None, None]


def _square_distance(a, b):
    return (jnp.sum(a * a, -1)[:, :, None] + jnp.sum(b * b, -1)[:, None, :]
            - 2.0 * jnp.einsum('bnd,bmd->bnm', a, b))


def _gather(feat, idx):
    return jax.vmap(lambda f, i: f[:, i])(feat, idx)


def _gather_nbr(feat, idx):
    b, c, ns = feat.shape
    _, nq, k = idx.shape
    ft = jnp.transpose(feat, (0, 2, 1)).reshape(b * ns, c)
    gidx = (idx + (jnp.arange(b, dtype=idx.dtype) * ns)[:, None, None]).reshape(-1)
    rows = jnp.take(ft, gidx, axis=0)
    return jnp.transpose(rows.reshape(b, nq, k, c), (0, 3, 1, 2))


def _three_inter(f, p1, p2):
    idx, val = _knn_topk(jnp.transpose(p1, (0, 2, 1)), p2, 3)
    dis = jnp.transpose(val, (0, 2, 1))
    dist_recip = 1.0 / (dis + 1e-8)
    weight = dist_recip / jnp.sum(dist_recip, axis=2, keepdims=True)
    nb = _gather_nbr(f, jnp.transpose(idx, (0, 2, 1)))
    return jnp.sum(nb * weight[:, None, :, :], axis=-1)


def _mlp_res(x, p):
    shortcut = _conv1d(x, p['ws'], p['bs'])
    h = jax.nn.relu(_conv1d(x, p['w1'], p['b1']))
    return _conv1d(h, p['w2'], p['b2']) + shortcut


def _vector_attention(p, pq, fq, ps, fs, n_knn):
    identity = fq
    q = _conv1d(fq, p['wq'], p['bq'])
    k = _conv1d(fs, p['wk'], p['bk'])
    v = _conv1d(fs, p['wv'], p['bv'])
    idx_t, _ = _knn_topk(jnp.transpose(ps, (0, 2, 1)), pq, n_knn)
    idx = jnp.transpose(idx_t, (0, 2, 1))
    g = _gather_nbr(jnp.concatenate([k, v, ps], axis=1), idx)
    key = g[:, :64]
    vg = g[:, 64:128]
    qk_rel = q[:, :, :, None] - key
    pos_rel = pq[:, :, :, None] - g[:, 128:131]
    pe = _conv2d(pos_rel, p['pw1'], p['pb1'])
    pe = jax.nn.relu(_bn2d(pe, p['pg1'], p['pbt1']))
    pe = _conv2d(pe, p['pw2'], p['pb2'])
    a = _conv2d(qk_rel + pe, p['aw1'], p['ab1'])
    a = jax.nn.relu(_bn2d(a, p['ag1'], p['abt1']))
    a = _conv2d(a, p['aw2'], p['ab2'])
    a = jax.nn.softmax(a, axis=-1)
    val = vg + pe
    agg = jnp.sum(a * val, axis=-1)
    return _conv1d(agg, p['we'], p['be']) + identity


def kernel(pq, fq, ps, fs, params):
    b, _, n = pq.shape
    xyz_all = jnp.concatenate(
        [jnp.transpose(pq, (0, 2, 1)), jnp.transpose(ps, (0, 2, 1))], axis=0)
    idx512 = _fps_batched(xyz_all, n // 4)
    fq512, fs512 = idx512[:b], idx512[b:]
    fq256, fs256 = fq512[:, :n // 8], fs512[:, :n // 8]
    fps_q = [None, fq512, fq256]
    fps_s = [None, fs512, fs256]

    num_scale = len(_DOWN_RATES)
    pre_f = None
    pre_pos = None
    for i in range(num_scale - 1, -1, -1):
        pos1 = pq if fps_q[i] is None else _gather(pq, fps_q[i])
        pos2 = ps if fps_s[i] is None else _gather(ps, fps_s[i])
        f1 = fq if fps_q[i] is None else _gather(fq, fps_q[i])
        f2 = fs if fps_s[i] is None else _gather(fs, fps_s[i])
        if i != num_scale - 1:
            proj1 = _three_inter(pre_f, pre_pos, pos1)
            proj2 = _three_inter(pre_f, pre_pos, pos2)
            f1 = _mlp_res(jnp.concatenate([f1, proj1], axis=1), params['qmlp'][i])
            f2 = _mlp_res(jnp.concatenate([f2, proj2], axis=1), params['smlp'][i])
        pre_f = _vector_attention(params['attn'][i], pos1, f1, pos2, f2, _KNNS[i])
        pre_pos = pos1
    return (pre_f, fps_q[1], fps_q[2], fps_s[1], fps_s[2])
